# Initial kernel scaffold; baseline (speedup 1.0000x reference)
import functools

import jax
import jax.numpy as jnp
from jax import lax
from jax.experimental import pallas as pl
from jax.experimental.pallas import tpu as pltpu
from jax.experimental.pallas import tpu_sc as plsc

N_NODES = 10000
ROWS = 10240
NF = 128
HALF = 64
NCLS = 40
NCORES = 2
NSUB = 16
EB = 128
TILE_ROWS = ROWS // NSUB
BLK = 512
GRID = ROWS // BLK

_f32 = jnp.float32


def _sc_mesh():
  return plsc.VectorSubcoreMesh(
      core_axis_name="c", subcore_axis_name="s",
      num_cores=NCORES, num_subcores=NSUB)


@functools.lru_cache(maxsize=None)
def _deg_kernel(nb1, nb2):
  @functools.partial(
      pl.kernel,
      mesh=_sc_mesh(),
      out_type=[jax.ShapeDtypeStruct((ROWS, 16), _f32),
                jax.ShapeDtypeStruct((ROWS, 16), _f32)],
      scratch_types=[pltpu.VMEM((EB,), jnp.int32),
                     pltpu.VMEM((EB, 16), _f32),
                     pltpu.VMEM_SHARED((ROWS, 16), _f32)],
  )
  def deg_k(dst1, dst2, ones_h, zeros_h, deg1, deg2, didx, ones_v, dacc):
    c = lax.axis_index("c")
    s = lax.axis_index("s")
    sl = pl.ds(s * TILE_ROWS, TILE_ROWS)
    pltpu.sync_copy(ones_h, ones_v)
    pltpu.sync_copy(zeros_h.at[sl], dacc.at[sl])
    plsc.subcore_barrier()

    def run(dst_h, nb):
      def body(j, carry):
        off = (s * nb + j) * EB
        pltpu.sync_copy(dst_h.at[pl.ds(off, EB)], didx)
        pltpu.sync_copy(ones_v, dacc.at[didx], add=True)
        return carry
      lax.fori_loop(0, nb, body, 0)

    @pl.when(c == 0)
    def _():
      run(dst1, nb1)

    @pl.when(c == 1)
    def _():
      run(dst2, nb2)

    plsc.subcore_barrier()

    @pl.when(c == 0)
    def _():
      pltpu.sync_copy(dacc.at[sl], deg1.at[sl])

    @pl.when(c == 1)
    def _():
      pltpu.sync_copy(dacc.at[sl], deg2.at[sl])

  return deg_k


@functools.lru_cache(maxsize=None)
def _spmm_kernel(nb):
  @functools.partial(
      pl.kernel,
      mesh=_sc_mesh(),
      out_type=[jax.ShapeDtypeStruct((ROWS, HALF), _f32),
                jax.ShapeDtypeStruct((ROWS, HALF), _f32)],
      scratch_types=[pltpu.VMEM((EB,), jnp.int32),
                     pltpu.VMEM((EB,), jnp.int32),
                     pltpu.VMEM((EB, HALF), _f32),
                     pltpu.VMEM_SHARED((ROWS, HALF), _f32)],
  )
  def spmm_k(src_h, dst_h, ya_h, yb_h, za_h, zb_h, sidx, didx, rows, acc):
    c = lax.axis_index("c")
    s = lax.axis_index("s")
    sl = pl.ds(s * TILE_ROWS, TILE_ROWS)

    @pl.when(c == 0)
    def _():
      pltpu.sync_copy(ya_h.at[sl], acc.at[sl])

    @pl.when(c == 1)
    def _():
      pltpu.sync_copy(yb_h.at[sl], acc.at[sl])

    plsc.subcore_barrier()

    def run(y_h):
      def body(j, carry):
        off = (s * nb + j) * EB
        pltpu.sync_copy(src_h.at[pl.ds(off, EB)], sidx)
        pltpu.sync_copy(dst_h.at[pl.ds(off, EB)], didx)
        pltpu.sync_copy(y_h.at[sidx], rows)
        pltpu.sync_copy(rows, acc.at[didx], add=True)
        return carry
      lax.fori_loop(0, nb, body, 0)

    @pl.when(c == 0)
    def _():
      run(ya_h)

    @pl.when(c == 1)
    def _():
      run(yb_h)

    plsc.subcore_barrier()

    @pl.when(c == 0)
    def _():
      pltpu.sync_copy(acc.at[sl], za_h.at[sl])

    @pl.when(c == 1)
    def _():
      pltpu.sync_copy(acc.at[sl], zb_h.at[sl])

  return spmm_k


def _dinv(deg_blk):
  return lax.rsqrt(deg_blk[:, 0:1] + 1.0)


def _k1_body(h_ref, w_ref, b_ref, d1_ref, d2_ref,
             x_ref, y1a_ref, y1b_ref, y2a_ref, y2b_ref):
  xb = lax.dot_general(h_ref[...], w_ref[...], (((1,), (1,)), ((), ())))
  xb = jnp.maximum(xb + b_ref[...], 0.0)
  x_ref[...] = xb
  y1 = xb * _dinv(d1_ref)
  y2 = xb * _dinv(d2_ref)
  y1a_ref[...] = y1[:, :HALF]
  y1b_ref[...] = y1[:, HALF:]
  y2a_ref[...] = y2[:, :HALF]
  y2b_ref[...] = y2[:, HALF:]


def _k1(hp, W1, b1r, deg1, deg2):
  blk = lambda r, cdim: pl.BlockSpec((r, cdim), lambda i: (i, 0))
  full = lambda shape: pl.BlockSpec(shape, lambda i: (0, 0))
  return pl.pallas_call(
      _k1_body,
      grid=(GRID,),
      in_specs=[blk(BLK, NF), full((NF, NF)), full((1, NF)),
                blk(BLK, 16), blk(BLK, 16)],
      out_specs=[blk(BLK, NF)] + [blk(BLK, HALF)] * 4,
      out_shape=[jax.ShapeDtypeStruct((ROWS, NF), _f32)] +
                [jax.ShapeDtypeStruct((ROWS, HALF), _f32)] * 4,
  )(hp, W1, b1r, deg1, deg2)


def _k2_body(za_ref, zb_ref, dk_ref, d1_ref, d2_ref,
             aa_ref, ab_ref, u1a_ref, u1b_ref, u2a_ref, u2b_ref):
  dk = _dinv(dk_ref)
  d1 = _dinv(d1_ref)
  d2 = _dinv(d2_ref)
  aa = za_ref[...] * dk
  ab = zb_ref[...] * dk
  aa_ref[...] = aa
  ab_ref[...] = ab
  u1a_ref[...] = aa * d1
  u1b_ref[...] = ab * d1
  u2a_ref[...] = aa * d2
  u2b_ref[...] = ab * d2


def _k2(za, zb, degk, deg1, deg2):
  blk = lambda cdim: pl.BlockSpec((BLK, cdim), lambda i: (i, 0))
  return pl.pallas_call(
      _k2_body,
      grid=(GRID,),
      in_specs=[blk(HALF), blk(HALF), blk(16), blk(16), blk(16)],
      out_specs=[blk(HALF)] * 6,
      out_shape=[jax.ShapeDtypeStruct((ROWS, HALF), _f32)] * 6,
  )(za, zb, degk, deg1, deg2)


def _k3_body(x_ref, a1a_ref, a1b_ref, a2a_ref, a2b_ref,
             z11a_ref, z11b_ref, z12a_ref, z12b_ref,
             z21a_ref, z21b_ref, z22a_ref, z22b_ref,
             d1_ref, d2_ref, w2_ref, b2_ref, out_ref):
  d1 = _dinv(d1_ref)
  d2 = _dinv(d2_ref)
  w2 = w2_ref[...]

  def mm(v, c0, w):
    return lax.dot_general(v, w2[:, c0:c0 + w], (((1,), (1,)), ((), ())))

  acc = b2_ref[...] + mm(x_ref[...], 0, NF)
  acc += mm(a1a_ref[...], 128, HALF)
  acc += mm(a1b_ref[...], 192, HALF)
  acc += mm(a2a_ref[...], 256, HALF)
  acc += mm(a2b_ref[...], 320, HALF)
  acc += mm(z11a_ref[...] * d1, 384, HALF)
  acc += mm(z11b_ref[...] * d1, 448, HALF)
  acc += mm(z12a_ref[...] * d1, 512, HALF)
  acc += mm(z12b_ref[...] * d1, 576, HALF)
  acc += mm(z21a_ref[...] * d2, 640, HALF)
  acc += mm(z21b_ref[...] * d2, 704, HALF)
  acc += mm(z22a_ref[...] * d2, 768, HALF)
  acc += mm(z22b_ref[...] * d2, 832, HALF)
  m = jnp.max(acc, axis=1, keepdims=True)
  lse = jnp.log(jnp.sum(jnp.exp(acc - m), axis=1, keepdims=True)) + m
  out_ref[...] = acc - lse


def _k3(x, a1a, a1b, a2a, a2b, zs, deg1, deg2, W2, b2r):
  blk = lambda cdim: pl.BlockSpec((BLK, cdim), lambda i: (i, 0))
  full = lambda shape: pl.BlockSpec(shape, lambda i: (0, 0))
  return pl.pallas_call(
      _k3_body,
      grid=(GRID,),
      in_specs=[blk(NF)] + [blk(HALF)] * 12 + [blk(16), blk(16),
                full((NCLS, 7 * NF)), full((1, NCLS))],
      out_specs=blk(NCLS),
      out_shape=jax.ShapeDtypeStruct((ROWS, NCLS), _f32),
  )(x, a1a, a1b, a2a, a2b, *zs, deg1, deg2, W2, b2r)


def _pad_edges(ei, total):
  pad = total - ei.shape[1]
  src = jnp.concatenate([ei[0], jnp.zeros((pad,), jnp.int32)])
  dst = jnp.concatenate([ei[1], jnp.full((pad,), N_NODES, jnp.int32)])
  return src, dst


def _round_up(e, m):
  return ((e + m - 1) // m) * m


def kernel(h, edge_index, edge_index2, W1, b1, W2, b2):
  chunk = NSUB * EB
  e1p = _round_up(edge_index.shape[1], chunk)
  e2p = _round_up(edge_index2.shape[1], chunk)
  nb1 = e1p // chunk
  nb2 = e2p // chunk
  src1, dst1 = _pad_edges(edge_index, e1p)
  src2, dst2 = _pad_edges(edge_index2, e2p)

  ones_h = jnp.ones((EB, 16), _f32)
  zeros_h = jnp.zeros((ROWS, 16), _f32)
  deg1, deg2 = _deg_kernel(nb1, nb2)(dst1, dst2, ones_h, zeros_h)

  hp = jnp.zeros((ROWS, NF), _f32).at[:N_NODES].set(h)
  b1r = b1.reshape(1, NF)
  b2r = b2.reshape(1, NCLS)

  x, y1a, y1b, y2a, y2b = _k1(hp, W1, b1r, deg1, deg2)

  z1a, z1b = _spmm_kernel(nb1)(src1, dst1, y1a, y1b)
  z2a, z2b = _spmm_kernel(nb2)(src2, dst2, y2a, y2b)

  a1a, a1b, u11a, u11b, u21a, u21b = _k2(z1a, z1b, deg1, deg1, deg2)
  a2a, a2b, u12a, u12b, u22a, u22b = _k2(z2a, z2b, deg2, deg1, deg2)

  z11 = _spmm_kernel(nb1)(src1, dst1, u11a, u11b)
  z12 = _spmm_kernel(nb1)(src1, dst1, u12a, u12b)
  z21 = _spmm_kernel(nb2)(src2, dst2, u21a, u21b)
  z22 = _spmm_kernel(nb2)(src2, dst2, u22a, u22b)
  zs = (*z11, *z12, *z21, *z22)

  out = _k3(x, a1a, a1b, a2a, a2b, zs, deg1, deg2, W2, b2r)
  return out[:N_NODES]


# speedup vs baseline: 7.8978x; 7.8978x over previous
import functools

import jax
import jax.numpy as jnp
from jax import lax
from jax.experimental import pallas as pl
from jax.experimental.pallas import tpu as pltpu
from jax.experimental.pallas import tpu_sc as plsc

N_NODES = 10000
ROWS = 10240
NF = 128
HALF = 64
NCLS = 40
NCORES = 2
NSUB = 16
EB = 128
TILE_ROWS = ROWS // NSUB
BLK = 512
GRID = ROWS // BLK

_f32 = jnp.float32


def _sc_mesh():
  return plsc.VectorSubcoreMesh(
      core_axis_name="c", subcore_axis_name="s",
      num_cores=NCORES, num_subcores=NSUB)


@functools.lru_cache(maxsize=None)
def _deg_kernel(nb1, nb2):
  @functools.partial(
      pl.kernel,
      mesh=_sc_mesh(),
      out_type=[jax.ShapeDtypeStruct((ROWS, 16), _f32),
                jax.ShapeDtypeStruct((ROWS, 16), _f32)],
      scratch_types=[pltpu.VMEM((EB,), jnp.int32),
                     pltpu.VMEM((EB, 16), _f32),
                     pltpu.VMEM_SHARED((ROWS, 16), _f32)],
      compiler_params=pltpu.CompilerParams(use_tc_tiling_on_sc=False),
  )
  def deg_k(dst1, dst2, ones_h, zeros_h, deg1, deg2, didx, ones_v, dacc):
    c = lax.axis_index("c")
    s = lax.axis_index("s")
    sl = pl.ds(s * TILE_ROWS, TILE_ROWS)
    pltpu.sync_copy(ones_h, ones_v)
    pltpu.sync_copy(zeros_h.at[sl], dacc.at[sl])
    plsc.subcore_barrier()

    def run(dst_h, nb):
      def body(j, carry):
        off = (s * nb + j) * EB
        pltpu.sync_copy(dst_h.at[pl.ds(off, EB)], didx)
        pltpu.sync_copy(ones_v, dacc.at[didx], add=True)
        return carry
      lax.fori_loop(0, nb, body, 0)

    @pl.when(c == 0)
    def _():
      run(dst1, nb1)

    @pl.when(c == 1)
    def _():
      run(dst2, nb2)

    plsc.subcore_barrier()

    @pl.when(c == 0)
    def _():
      pltpu.sync_copy(dacc.at[sl], deg1.at[sl])

    @pl.when(c == 1)
    def _():
      pltpu.sync_copy(dacc.at[sl], deg2.at[sl])

  return deg_k


@functools.lru_cache(maxsize=None)
def _spmm_kernel(nb):
  @functools.partial(
      pl.kernel,
      mesh=_sc_mesh(),
      out_type=[jax.ShapeDtypeStruct((ROWS, HALF), _f32),
                jax.ShapeDtypeStruct((ROWS, HALF), _f32)],
      scratch_types=[pltpu.VMEM((EB,), jnp.int32),
                     pltpu.VMEM((EB,), jnp.int32),
                     pltpu.VMEM((EB, HALF), _f32),
                     pltpu.VMEM_SHARED((ROWS, HALF), _f32)],
      compiler_params=pltpu.CompilerParams(use_tc_tiling_on_sc=False),
  )
  def spmm_k(src_h, dst_h, ya_h, yb_h, za_h, zb_h, sidx, didx, rows, acc):
    c = lax.axis_index("c")
    s = lax.axis_index("s")
    sl = pl.ds(s * TILE_ROWS, TILE_ROWS)

    @pl.when(c == 0)
    def _():
      pltpu.sync_copy(ya_h.at[sl], acc.at[sl])

    @pl.when(c == 1)
    def _():
      pltpu.sync_copy(yb_h.at[sl], acc.at[sl])

    plsc.subcore_barrier()

    def run(y_h):
      def body(j, carry):
        off = (s * nb + j) * EB
        pltpu.sync_copy(src_h.at[pl.ds(off, EB)], sidx)
        pltpu.sync_copy(dst_h.at[pl.ds(off, EB)], didx)
        pltpu.sync_copy(y_h.at[sidx], rows)
        pltpu.sync_copy(rows, acc.at[didx], add=True)
        return carry
      lax.fori_loop(0, nb, body, 0)

    @pl.when(c == 0)
    def _():
      run(ya_h)

    @pl.when(c == 1)
    def _():
      run(yb_h)

    plsc.subcore_barrier()

    @pl.when(c == 0)
    def _():
      pltpu.sync_copy(acc.at[sl], za_h.at[sl])

    @pl.when(c == 1)
    def _():
      pltpu.sync_copy(acc.at[sl], zb_h.at[sl])

  return spmm_k


def _dinv(deg_blk):
  return lax.rsqrt(deg_blk[:, 0:1] + 1.0)


def _k1_body(h_ref, w_ref, b_ref, d1_ref, d2_ref,
             x_ref, y1a_ref, y1b_ref, y2a_ref, y2b_ref):
  xb = lax.dot_general(h_ref[...], w_ref[...], (((1,), (1,)), ((), ())))
  xb = jnp.maximum(xb + b_ref[...], 0.0)
  x_ref[...] = xb
  y1 = xb * _dinv(d1_ref)
  y2 = xb * _dinv(d2_ref)
  y1a_ref[...] = y1[:, :HALF]
  y1b_ref[...] = y1[:, HALF:]
  y2a_ref[...] = y2[:, :HALF]
  y2b_ref[...] = y2[:, HALF:]


def _k1(hp, W1, b1r, deg1, deg2):
  blk = lambda r, cdim: pl.BlockSpec((r, cdim), lambda i: (i, 0))
  full = lambda shape: pl.BlockSpec(shape, lambda i: (0, 0))
  return pl.pallas_call(
      _k1_body,
      grid=(GRID,),
      in_specs=[blk(BLK, NF), full((NF, NF)), full((1, NF)),
                blk(BLK, 16), blk(BLK, 16)],
      out_specs=[blk(BLK, NF)] + [blk(BLK, HALF)] * 4,
      out_shape=[jax.ShapeDtypeStruct((ROWS, NF), _f32)] +
                [jax.ShapeDtypeStruct((ROWS, HALF), _f32)] * 4,
  )(hp, W1, b1r, deg1, deg2)


def _k2_body(za_ref, zb_ref, dk_ref, d1_ref, d2_ref,
             aa_ref, ab_ref, u1a_ref, u1b_ref, u2a_ref, u2b_ref):
  dk = _dinv(dk_ref)
  d1 = _dinv(d1_ref)
  d2 = _dinv(d2_ref)
  aa = za_ref[...] * dk
  ab = zb_ref[...] * dk
  aa_ref[...] = aa
  ab_ref[...] = ab
  u1a_ref[...] = aa * d1
  u1b_ref[...] = ab * d1
  u2a_ref[...] = aa * d2
  u2b_ref[...] = ab * d2


def _k2(za, zb, degk, deg1, deg2):
  blk = lambda cdim: pl.BlockSpec((BLK, cdim), lambda i: (i, 0))
  return pl.pallas_call(
      _k2_body,
      grid=(GRID,),
      in_specs=[blk(HALF), blk(HALF), blk(16), blk(16), blk(16)],
      out_specs=[blk(HALF)] * 6,
      out_shape=[jax.ShapeDtypeStruct((ROWS, HALF), _f32)] * 6,
  )(za, zb, degk, deg1, deg2)


def _k3_body(x_ref, a1a_ref, a1b_ref, a2a_ref, a2b_ref,
             z11a_ref, z11b_ref, z12a_ref, z12b_ref,
             z21a_ref, z21b_ref, z22a_ref, z22b_ref,
             d1_ref, d2_ref, w2_ref, b2_ref, out_ref):
  d1 = _dinv(d1_ref)
  d2 = _dinv(d2_ref)
  w2 = w2_ref[...]

  def mm(v, c0, w):
    return lax.dot_general(v, w2[:, c0:c0 + w], (((1,), (1,)), ((), ())))

  acc = b2_ref[...] + mm(x_ref[...], 0, NF)
  acc += mm(a1a_ref[...], 128, HALF)
  acc += mm(a1b_ref[...], 192, HALF)
  acc += mm(a2a_ref[...], 256, HALF)
  acc += mm(a2b_ref[...], 320, HALF)
  acc += mm(z11a_ref[...] * d1, 384, HALF)
  acc += mm(z11b_ref[...] * d1, 448, HALF)
  acc += mm(z12a_ref[...] * d1, 512, HALF)
  acc += mm(z12b_ref[...] * d1, 576, HALF)
  acc += mm(z21a_ref[...] * d2, 640, HALF)
  acc += mm(z21b_ref[...] * d2, 704, HALF)
  acc += mm(z22a_ref[...] * d2, 768, HALF)
  acc += mm(z22b_ref[...] * d2, 832, HALF)
  m = jnp.max(acc, axis=1, keepdims=True)
  lse = jnp.log(jnp.sum(jnp.exp(acc - m), axis=1, keepdims=True)) + m
  out_ref[...] = acc - lse


def _k3(x, a1a, a1b, a2a, a2b, zs, deg1, deg2, W2, b2r):
  blk = lambda cdim: pl.BlockSpec((BLK, cdim), lambda i: (i, 0))
  full = lambda shape: pl.BlockSpec(shape, lambda i: (0, 0))
  return pl.pallas_call(
      _k3_body,
      grid=(GRID,),
      in_specs=[blk(NF)] + [blk(HALF)] * 12 + [blk(16), blk(16),
                full((NCLS, 7 * NF)), full((1, NCLS))],
      out_specs=blk(NCLS),
      out_shape=jax.ShapeDtypeStruct((ROWS, NCLS), _f32),
  )(x, a1a, a1b, a2a, a2b, *zs, deg1, deg2, W2, b2r)


def _pad_edges(ei, total):
  pad = total - ei.shape[1]
  src = jnp.concatenate([ei[0], jnp.zeros((pad,), jnp.int32)])
  dst = jnp.concatenate([ei[1], jnp.full((pad,), N_NODES, jnp.int32)])
  return src, dst


def _round_up(e, m):
  return ((e + m - 1) // m) * m


def kernel(h, edge_index, edge_index2, W1, b1, W2, b2):
  chunk = NSUB * EB
  e1p = _round_up(edge_index.shape[1], chunk)
  e2p = _round_up(edge_index2.shape[1], chunk)
  nb1 = e1p // chunk
  nb2 = e2p // chunk
  src1, dst1 = _pad_edges(edge_index, e1p)
  src2, dst2 = _pad_edges(edge_index2, e2p)

  ones_h = jnp.ones((EB, 16), _f32)
  zeros_h = jnp.zeros((ROWS, 16), _f32)
  deg1, deg2 = _deg_kernel(nb1, nb2)(dst1, dst2, ones_h, zeros_h)

  hp = jnp.zeros((ROWS, NF), _f32).at[:N_NODES].set(h)
  b1r = b1.reshape(1, NF)
  b2r = b2.reshape(1, NCLS)

  x, y1a, y1b, y2a, y2b = _k1(hp, W1, b1r, deg1, deg2)

  z1a, z1b = _spmm_kernel(nb1)(src1, dst1, y1a, y1b)
  z2a, z2b = _spmm_kernel(nb2)(src2, dst2, y2a, y2b)

  a1a, a1b, u11a, u11b, u21a, u21b = _k2(z1a, z1b, deg1, deg1, deg2)
  a2a, a2b, u12a, u12b, u22a, u22b = _k2(z2a, z2b, deg2, deg1, deg2)

  z11 = _spmm_kernel(nb1)(src1, dst1, u11a, u11b)
  z12 = _spmm_kernel(nb1)(src1, dst1, u12a, u12b)
  z21 = _spmm_kernel(nb2)(src2, dst2, u21a, u21b)
  z22 = _spmm_kernel(nb2)(src2, dst2, u22a, u22b)
  zs = (*z11, *z12, *z21, *z22)

  out = _k3(x, a1a, a1b, a2a, a2b, zs, deg1, deg2, W2, b2r)
  return out[:N_NODES]


# speedup vs baseline: 19.7120x; 2.4959x over previous
import functools

import jax
import jax.numpy as jnp
from jax import lax
from jax.experimental import pallas as pl
from jax.experimental.pallas import tpu as pltpu
from jax.experimental.pallas import tpu_sc as plsc

N_NODES = 10000
ROWS = 10240
NF = 128
HALF = 64
NCLS = 40
NCORES = 2
NSUB = 16
EB = 128
SB = 8
TILE_ROWS = ROWS // NSUB
BLK = 512
GRID = ROWS // BLK

_f32 = jnp.float32


def _sc_mesh():
  return plsc.VectorSubcoreMesh(
      core_axis_name="c", subcore_axis_name="s",
      num_cores=NCORES, num_subcores=NSUB)


@functools.lru_cache(maxsize=None)
def _deg_kernel(nb1, nb2):
  @functools.partial(
      pl.kernel,
      mesh=_sc_mesh(),
      out_type=[jax.ShapeDtypeStruct((ROWS, 16), _f32),
                jax.ShapeDtypeStruct((ROWS, 16), _f32)],
      scratch_types=[pltpu.VMEM((SB, EB), jnp.int32),
                     pltpu.VMEM((EB, 16), _f32),
                     pltpu.VMEM_SHARED((ROWS, 16), _f32),
                     pltpu.SemaphoreType.DMA],
      compiler_params=pltpu.CompilerParams(use_tc_tiling_on_sc=False),
  )
  def deg_k(dst1, dst2, ones_h, zeros_h, deg1, deg2, didx, ones_v, dacc, sem):
    c = lax.axis_index("c")
    s = lax.axis_index("s")
    sl = pl.ds(s * TILE_ROWS, TILE_ROWS)
    pltpu.sync_copy(ones_h, ones_v)
    pltpu.sync_copy(zeros_h.at[sl], dacc.at[sl])
    plsc.subcore_barrier()

    def run(dst_h, nb):
      def chunk(cc, carry):
        pltpu.sync_copy(dst_h.at[pl.ds(s * nb + cc * SB, SB)], didx)
        descs = [pltpu.async_copy(ones_v, dacc.at[didx.at[b]], sem, add=True)
                 for b in range(SB)]
        for d in descs:
          d.wait()
        return carry
      lax.fori_loop(0, nb // SB, chunk, 0)

    @pl.when(c == 0)
    def _():
      run(dst1, nb1)

    @pl.when(c == 1)
    def _():
      run(dst2, nb2)

    plsc.subcore_barrier()

    @pl.when(c == 0)
    def _():
      pltpu.sync_copy(dacc.at[sl], deg1.at[sl])

    @pl.when(c == 1)
    def _():
      pltpu.sync_copy(dacc.at[sl], deg2.at[sl])

  return deg_k


def _run_pipe(s, nb, src_h, dst_h, y_h, acc, sidx, didx, rows, sems):
  def chunk(cc, carry):
    base = s * nb + cc * SB
    pltpu.sync_copy(src_h.at[pl.ds(base, SB)], sidx)
    pltpu.sync_copy(dst_h.at[pl.ds(base, SB)], didx)
    descs = {}
    descs[0] = pltpu.async_copy(y_h.at[sidx.at[0]], rows.at[0], sems[0])
    for b in range(SB):
      if b + 1 < SB:
        p = (b + 1) % 2
        descs[b + 1] = pltpu.async_copy(y_h.at[sidx.at[b + 1]], rows.at[p],
                                        sems[p])
      descs[b].wait()
      pltpu.sync_copy(rows.at[b % 2], acc.at[didx.at[b]], add=True)
    return carry
  lax.fori_loop(0, nb // SB, chunk, 0)


def _init_acc(c, sl, ya_h, yb_h, acc):
  @pl.when(c == 0)
  def _():
    pltpu.sync_copy(ya_h.at[sl], acc.at[sl])

  @pl.when(c == 1)
  def _():
    pltpu.sync_copy(yb_h.at[sl], acc.at[sl])


def _copy_out(c, sl, acc, za_h, zb_h):
  @pl.when(c == 0)
  def _():
    pltpu.sync_copy(acc.at[sl], za_h.at[sl])

  @pl.when(c == 1)
  def _():
    pltpu.sync_copy(acc.at[sl], zb_h.at[sl])


@functools.lru_cache(maxsize=None)
def _stage1_kernel(nb1, nb2):
  @functools.partial(
      pl.kernel,
      mesh=_sc_mesh(),
      out_type=[jax.ShapeDtypeStruct((ROWS, HALF), _f32)] * 4,
      scratch_types=[pltpu.VMEM((SB, EB), jnp.int32),
                     pltpu.VMEM((SB, EB), jnp.int32),
                     pltpu.VMEM((2, EB, HALF), _f32),
                     pltpu.VMEM_SHARED((ROWS, HALF), _f32),
                     pltpu.VMEM_SHARED((ROWS, HALF), _f32),
                     pltpu.SemaphoreType.DMA,
                     pltpu.SemaphoreType.DMA],
      compiler_params=pltpu.CompilerParams(use_tc_tiling_on_sc=False),
  )
  def k(src1, dst1, src2, dst2, y1a, y1b, y2a, y2b,
        z1a, z1b, z2a, z2b, sidx, didx, rows, acc1, acc2, sem0, sem1):
    c = lax.axis_index("c")
    s = lax.axis_index("s")
    sl = pl.ds(s * TILE_ROWS, TILE_ROWS)
    _init_acc(c, sl, y1a, y1b, acc1)
    _init_acc(c, sl, y2a, y2b, acc2)
    plsc.subcore_barrier()

    @pl.when(c == 0)
    def _():
      _run_pipe(s, nb1, src1, dst1, y1a, acc1, sidx, didx, rows, (sem0, sem1))
      _run_pipe(s, nb2, src2, dst2, y2a, acc2, sidx, didx, rows, (sem0, sem1))

    @pl.when(c == 1)
    def _():
      _run_pipe(s, nb1, src1, dst1, y1b, acc1, sidx, didx, rows, (sem0, sem1))
      _run_pipe(s, nb2, src2, dst2, y2b, acc2, sidx, didx, rows, (sem0, sem1))

    plsc.subcore_barrier()
    _copy_out(c, sl, acc1, z1a, z1b)
    _copy_out(c, sl, acc2, z2a, z2b)

  return k


@functools.lru_cache(maxsize=None)
def _stage2_kernel(nb):
  @functools.partial(
      pl.kernel,
      mesh=_sc_mesh(),
      out_type=[jax.ShapeDtypeStruct((ROWS, HALF), _f32)] * 4,
      scratch_types=[pltpu.VMEM((SB, EB), jnp.int32),
                     pltpu.VMEM((SB, EB), jnp.int32),
                     pltpu.VMEM((2, EB, HALF), _f32),
                     pltpu.VMEM((2, EB, HALF), _f32),
                     pltpu.VMEM_SHARED((ROWS, HALF), _f32),
                     pltpu.VMEM_SHARED((ROWS, HALF), _f32),
                     pltpu.SemaphoreType.DMA,
                     pltpu.SemaphoreType.DMA,
                     pltpu.SemaphoreType.DMA,
                     pltpu.SemaphoreType.DMA],
      compiler_params=pltpu.CompilerParams(use_tc_tiling_on_sc=False),
  )
  def k(src_h, dst_h, ua, ub, va, vb, zua, zub, zva, zvb,
        sidx, didx, rowsu, rowsv, accu, accv, su0, su1, sv0, sv1):
    c = lax.axis_index("c")
    s = lax.axis_index("s")
    sl = pl.ds(s * TILE_ROWS, TILE_ROWS)
    _init_acc(c, sl, ua, ub, accu)
    _init_acc(c, sl, va, vb, accv)
    plsc.subcore_barrier()

    def dual(u_h, v_h):
      def chunk(cc, carry):
        base = s * nb + cc * SB
        pltpu.sync_copy(src_h.at[pl.ds(base, SB)], sidx)
        pltpu.sync_copy(dst_h.at[pl.ds(base, SB)], didx)
        du = {0: pltpu.async_copy(u_h.at[sidx.at[0]], rowsu.at[0], su0)}
        dv = {0: pltpu.async_copy(v_h.at[sidx.at[0]], rowsv.at[0], sv0)}
        for b in range(SB):
          if b + 1 < SB:
            p = (b + 1) % 2
            du[b + 1] = pltpu.async_copy(u_h.at[sidx.at[b + 1]], rowsu.at[p],
                                         su0 if p == 0 else su1)
            dv[b + 1] = pltpu.async_copy(v_h.at[sidx.at[b + 1]], rowsv.at[p],
                                         sv0 if p == 0 else sv1)
          du[b].wait()
          pltpu.sync_copy(rowsu.at[b % 2], accu.at[didx.at[b]], add=True)
          dv[b].wait()
          pltpu.sync_copy(rowsv.at[b % 2], accv.at[didx.at[b]], add=True)
        return carry
      lax.fori_loop(0, nb // SB, chunk, 0)

    @pl.when(c == 0)
    def _():
      dual(ua, va)

    @pl.when(c == 1)
    def _():
      dual(ub, vb)

    plsc.subcore_barrier()
    _copy_out(c, sl, accu, zua, zub)
    _copy_out(c, sl, accv, zva, zvb)

  return k


def _dinv(deg_blk):
  return lax.rsqrt(deg_blk[:, 0:1] + 1.0)


def _k1_body(h_ref, w_ref, b_ref, d1_ref, d2_ref,
             x_ref, y1a_ref, y1b_ref, y2a_ref, y2b_ref):
  xb = lax.dot_general(h_ref[...], w_ref[...], (((1,), (1,)), ((), ())))
  xb = jnp.maximum(xb + b_ref[...], 0.0)
  x_ref[...] = xb
  y1 = xb * _dinv(d1_ref)
  y2 = xb * _dinv(d2_ref)
  y1a_ref[...] = y1[:, :HALF]
  y1b_ref[...] = y1[:, HALF:]
  y2a_ref[...] = y2[:, :HALF]
  y2b_ref[...] = y2[:, HALF:]


def _k1(hp, W1, b1r, deg1, deg2):
  blk = lambda r, cdim: pl.BlockSpec((r, cdim), lambda i: (i, 0))
  full = lambda shape: pl.BlockSpec(shape, lambda i: (0, 0))
  return pl.pallas_call(
      _k1_body,
      grid=(GRID,),
      in_specs=[blk(BLK, NF), full((NF, NF)), full((1, NF)),
                blk(BLK, 16), blk(BLK, 16)],
      out_specs=[blk(BLK, NF)] + [blk(BLK, HALF)] * 4,
      out_shape=[jax.ShapeDtypeStruct((ROWS, NF), _f32)] +
                [jax.ShapeDtypeStruct((ROWS, HALF), _f32)] * 4,
  )(hp, W1, b1r, deg1, deg2)


def _k2_body(za_ref, zb_ref, dk_ref, d1_ref, d2_ref,
             aa_ref, ab_ref, u1a_ref, u1b_ref, u2a_ref, u2b_ref):
  dk = _dinv(dk_ref)
  d1 = _dinv(d1_ref)
  d2 = _dinv(d2_ref)
  aa = za_ref[...] * dk
  ab = zb_ref[...] * dk
  aa_ref[...] = aa
  ab_ref[...] = ab
  u1a_ref[...] = aa * d1
  u1b_ref[...] = ab * d1
  u2a_ref[...] = aa * d2
  u2b_ref[...] = ab * d2


def _k2(za, zb, degk, deg1, deg2):
  blk = lambda cdim: pl.BlockSpec((BLK, cdim), lambda i: (i, 0))
  return pl.pallas_call(
      _k2_body,
      grid=(GRID,),
      in_specs=[blk(HALF), blk(HALF), blk(16), blk(16), blk(16)],
      out_specs=[blk(HALF)] * 6,
      out_shape=[jax.ShapeDtypeStruct((ROWS, HALF), _f32)] * 6,
  )(za, zb, degk, deg1, deg2)


def _k3_body(x_ref, a1a_ref, a1b_ref, a2a_ref, a2b_ref,
             z11a_ref, z11b_ref, z12a_ref, z12b_ref,
             z21a_ref, z21b_ref, z22a_ref, z22b_ref,
             d1_ref, d2_ref, w2_ref, b2_ref, out_ref):
  d1 = _dinv(d1_ref)
  d2 = _dinv(d2_ref)
  w2 = w2_ref[...]

  def mm(v, c0, w):
    return lax.dot_general(v, w2[:, c0:c0 + w], (((1,), (1,)), ((), ())))

  acc = b2_ref[...] + mm(x_ref[...], 0, NF)
  acc += mm(a1a_ref[...], 128, HALF)
  acc += mm(a1b_ref[...], 192, HALF)
  acc += mm(a2a_ref[...], 256, HALF)
  acc += mm(a2b_ref[...], 320, HALF)
  acc += mm(z11a_ref[...] * d1, 384, HALF)
  acc += mm(z11b_ref[...] * d1, 448, HALF)
  acc += mm(z12a_ref[...] * d1, 512, HALF)
  acc += mm(z12b_ref[...] * d1, 576, HALF)
  acc += mm(z21a_ref[...] * d2, 640, HALF)
  acc += mm(z21b_ref[...] * d2, 704, HALF)
  acc += mm(z22a_ref[...] * d2, 768, HALF)
  acc += mm(z22b_ref[...] * d2, 832, HALF)
  m = jnp.max(acc, axis=1, keepdims=True)
  lse = jnp.log(jnp.sum(jnp.exp(acc - m), axis=1, keepdims=True)) + m
  out_ref[...] = acc - lse


def _k3(x, a1a, a1b, a2a, a2b, zs, deg1, deg2, W2, b2r):
  blk = lambda cdim: pl.BlockSpec((BLK, cdim), lambda i: (i, 0))
  full = lambda shape: pl.BlockSpec(shape, lambda i: (0, 0))
  return pl.pallas_call(
      _k3_body,
      grid=(GRID,),
      in_specs=[blk(NF)] + [blk(HALF)] * 12 + [blk(16), blk(16),
                full((NCLS, 7 * NF)), full((1, NCLS))],
      out_specs=blk(NCLS),
      out_shape=jax.ShapeDtypeStruct((ROWS, NCLS), _f32),
  )(x, a1a, a1b, a2a, a2b, *zs, deg1, deg2, W2, b2r)


def _pad_edges(ei, total):
  pad = total - ei.shape[1]
  fill = jnp.arange(pad, dtype=jnp.int32)
  src = jnp.concatenate([ei[0], fill % N_NODES]).reshape(total // EB, EB)
  dst = jnp.concatenate(
      [ei[1], N_NODES + fill % (ROWS - N_NODES)]).reshape(total // EB, EB)
  return src, dst


def _round_up(e, m):
  return ((e + m - 1) // m) * m


def kernel(h, edge_index, edge_index2, W1, b1, W2, b2):
  chunk = NSUB * EB * SB
  e1p = _round_up(edge_index.shape[1], chunk)
  e2p = _round_up(edge_index2.shape[1], chunk)
  nb1 = e1p // (NSUB * EB)
  nb2 = e2p // (NSUB * EB)
  src1, dst1 = _pad_edges(edge_index, e1p)
  src2, dst2 = _pad_edges(edge_index2, e2p)

  ones_h = jnp.ones((EB, 16), _f32)
  zeros_h = jnp.zeros((ROWS, 16), _f32)
  deg1, deg2 = _deg_kernel(nb1, nb2)(dst1, dst2, ones_h, zeros_h)

  hp = jnp.zeros((ROWS, NF), _f32).at[:N_NODES].set(h)
  b1r = b1.reshape(1, NF)
  b2r = b2.reshape(1, NCLS)

  x, y1a, y1b, y2a, y2b = _k1(hp, W1, b1r, deg1, deg2)

  z1a, z1b, z2a, z2b = _stage1_kernel(nb1, nb2)(
      src1, dst1, src2, dst2, y1a, y1b, y2a, y2b)

  a1a, a1b, u11a, u11b, u21a, u21b = _k2(z1a, z1b, deg1, deg1, deg2)
  a2a, a2b, u12a, u12b, u22a, u22b = _k2(z2a, z2b, deg2, deg1, deg2)

  z11a, z11b, z12a, z12b = _stage2_kernel(nb1)(
      src1, dst1, u11a, u11b, u12a, u12b)
  z21a, z21b, z22a, z22b = _stage2_kernel(nb2)(
      src2, dst2, u21a, u21b, u22a, u22b)
  zs = (z11a, z11b, z12a, z12b, z21a, z21b, z22a, z22b)

  out = _k3(x, a1a, a1b, a2a, a2b, zs, deg1, deg2, W2, b2r)
  return out[:N_NODES]


# speedup vs baseline: 20.0848x; 1.0189x over previous
import functools

import jax
import jax.numpy as jnp
from jax import lax
from jax.experimental import pallas as pl
from jax.experimental.pallas import tpu as pltpu
from jax.experimental.pallas import tpu_sc as plsc

N_NODES = 10000
ROWS = 10240
NF = 128
HALF = 64
NCLS = 40
NCORES = 2
NSUB = 16
EB = 128
SB = 16
NBUF = 4
TILE_ROWS = ROWS // NSUB
BLK = 512
GRID = ROWS // BLK

_f32 = jnp.float32


def _sc_mesh():
  return plsc.VectorSubcoreMesh(
      core_axis_name="c", subcore_axis_name="s",
      num_cores=NCORES, num_subcores=NSUB)


@functools.lru_cache(maxsize=None)
def _deg_kernel(nb1, nb2):
  @functools.partial(
      pl.kernel,
      mesh=_sc_mesh(),
      out_type=[jax.ShapeDtypeStruct((ROWS, 16), _f32),
                jax.ShapeDtypeStruct((ROWS, 16), _f32)],
      scratch_types=[pltpu.VMEM((SB, EB), jnp.int32),
                     pltpu.VMEM((EB, 16), _f32),
                     pltpu.VMEM_SHARED((ROWS, 16), _f32),
                     pltpu.SemaphoreType.DMA],
      compiler_params=pltpu.CompilerParams(use_tc_tiling_on_sc=False),
  )
  def deg_k(dst1, dst2, ones_h, zeros_h, deg1, deg2, didx, ones_v, dacc, sem):
    c = lax.axis_index("c")
    s = lax.axis_index("s")
    sl = pl.ds(s * TILE_ROWS, TILE_ROWS)
    pltpu.sync_copy(ones_h, ones_v)
    pltpu.sync_copy(zeros_h.at[sl], dacc.at[sl])
    plsc.subcore_barrier()

    def run(dst_h, nb):
      def chunk(cc, carry):
        pltpu.sync_copy(dst_h.at[pl.ds(s * nb + cc * SB, SB)], didx)
        descs = [pltpu.async_copy(ones_v, dacc.at[didx.at[b]], sem, add=True)
                 for b in range(SB)]
        for d in descs:
          d.wait()
        return carry
      lax.fori_loop(0, nb // SB, chunk, 0)

    @pl.when(c == 0)
    def _():
      run(dst1, nb1)

    @pl.when(c == 1)
    def _():
      run(dst2, nb2)

    plsc.subcore_barrier()

    @pl.when(c == 0)
    def _():
      pltpu.sync_copy(dacc.at[sl], deg1.at[sl])

    @pl.when(c == 1)
    def _():
      pltpu.sync_copy(dacc.at[sl], deg2.at[sl])

  return deg_k


def _pipe_tables(s, nb, src_h, dst_h, tables, sidx, didx):
  def chunk(cc, carry):
    base = s * nb + cc * SB
    pltpu.sync_copy(src_h.at[pl.ds(base, SB)], sidx)
    pltpu.sync_copy(dst_h.at[pl.ds(base, SB)], didx)
    for (y_h, acc, rows, gsems, ssems) in tables:
      g = {}
      sd = {}
      for j in range(2):
        g[j] = pltpu.async_copy(y_h.at[sidx.at[j]], rows.at[j], gsems[j % 2])
      for b in range(SB):
        if b >= 2 and b + 2 < SB:
          sd[b - 2].wait()
        g[b].wait()
        if b + 2 < SB:
          g[b + 2] = pltpu.async_copy(
              y_h.at[sidx.at[b + 2]], rows.at[(b + 2) % NBUF], gsems[b % 2])
        sd[b] = pltpu.async_copy(rows.at[b % NBUF], acc.at[didx.at[b]],
                                 ssems[b % 2], add=True)
      for j in range(max(0, SB - 4), SB):
        sd[j].wait()
    return carry
  lax.fori_loop(0, nb // SB, chunk, 0)


def _init_acc(c, sl, ya_h, yb_h, acc):
  @pl.when(c == 0)
  def _():
    pltpu.sync_copy(ya_h.at[sl], acc.at[sl])

  @pl.when(c == 1)
  def _():
    pltpu.sync_copy(yb_h.at[sl], acc.at[sl])


def _copy_out(c, sl, acc, za_h, zb_h):
  @pl.when(c == 0)
  def _():
    pltpu.sync_copy(acc.at[sl], za_h.at[sl])

  @pl.when(c == 1)
  def _():
    pltpu.sync_copy(acc.at[sl], zb_h.at[sl])


@functools.lru_cache(maxsize=None)
def _stage1_kernel(nb1, nb2):
  @functools.partial(
      pl.kernel,
      mesh=_sc_mesh(),
      out_type=[jax.ShapeDtypeStruct((ROWS, HALF), _f32)] * 4,
      scratch_types=[pltpu.VMEM((SB, EB), jnp.int32),
                     pltpu.VMEM((SB, EB), jnp.int32),
                     pltpu.VMEM((NBUF, EB, HALF), _f32),
                     pltpu.VMEM_SHARED((ROWS, HALF), _f32),
                     pltpu.VMEM_SHARED((ROWS, HALF), _f32)] +
                    [pltpu.SemaphoreType.DMA] * 4,
      compiler_params=pltpu.CompilerParams(use_tc_tiling_on_sc=False),
  )
  def k(src1, dst1, src2, dst2, y1a, y1b, y2a, y2b,
        z1a, z1b, z2a, z2b, sidx, didx, rows, acc1, acc2, g0, g1, s0, s1):
    c = lax.axis_index("c")
    s = lax.axis_index("s")
    sl = pl.ds(s * TILE_ROWS, TILE_ROWS)
    _init_acc(c, sl, y1a, y1b, acc1)
    _init_acc(c, sl, y2a, y2b, acc2)
    plsc.subcore_barrier()

    @pl.when(c == 0)
    def _():
      _pipe_tables(s, nb1, src1, dst1,
                   [(y1a, acc1, rows, (g0, g1), (s0, s1))], sidx, didx)
      _pipe_tables(s, nb2, src2, dst2,
                   [(y2a, acc2, rows, (g0, g1), (s0, s1))], sidx, didx)

    @pl.when(c == 1)
    def _():
      _pipe_tables(s, nb1, src1, dst1,
                   [(y1b, acc1, rows, (g0, g1), (s0, s1))], sidx, didx)
      _pipe_tables(s, nb2, src2, dst2,
                   [(y2b, acc2, rows, (g0, g1), (s0, s1))], sidx, didx)

    plsc.subcore_barrier()
    _copy_out(c, sl, acc1, z1a, z1b)
    _copy_out(c, sl, acc2, z2a, z2b)

  return k


def _stage2_kernel(nb):
  k = _stage1_kernel(nb, nb)
  return lambda src, dst, ua, ub, va, vb: k(src, dst, src, dst, ua, ub, va, vb)


def _dinv(deg_blk):
  return lax.rsqrt(deg_blk[:, 0:1] + 1.0)


def _k1_body(h_ref, w_ref, b_ref, d1_ref, d2_ref,
             x_ref, y1a_ref, y1b_ref, y2a_ref, y2b_ref):
  xb = lax.dot_general(h_ref[...], w_ref[...], (((1,), (1,)), ((), ())))
  xb = jnp.maximum(xb + b_ref[...], 0.0)
  x_ref[...] = xb
  y1 = xb * _dinv(d1_ref)
  y2 = xb * _dinv(d2_ref)
  y1a_ref[...] = y1[:, :HALF]
  y1b_ref[...] = y1[:, HALF:]
  y2a_ref[...] = y2[:, :HALF]
  y2b_ref[...] = y2[:, HALF:]


def _k1(hp, W1, b1r, deg1, deg2):
  blk = lambda r, cdim: pl.BlockSpec((r, cdim), lambda i: (i, 0))
  full = lambda shape: pl.BlockSpec(shape, lambda i: (0, 0))
  return pl.pallas_call(
      _k1_body,
      grid=(GRID,),
      in_specs=[blk(BLK, NF), full((NF, NF)), full((1, NF)),
                blk(BLK, 16), blk(BLK, 16)],
      out_specs=[blk(BLK, NF)] + [blk(BLK, HALF)] * 4,
      out_shape=[jax.ShapeDtypeStruct((ROWS, NF), _f32)] +
                [jax.ShapeDtypeStruct((ROWS, HALF), _f32)] * 4,
  )(hp, W1, b1r, deg1, deg2)


def _k2_body(za_ref, zb_ref, dk_ref, d1_ref, d2_ref,
             aa_ref, ab_ref, u1a_ref, u1b_ref, u2a_ref, u2b_ref):
  dk = _dinv(dk_ref)
  d1 = _dinv(d1_ref)
  d2 = _dinv(d2_ref)
  aa = za_ref[...] * dk
  ab = zb_ref[...] * dk
  aa_ref[...] = aa
  ab_ref[...] = ab
  u1a_ref[...] = aa * d1
  u1b_ref[...] = ab * d1
  u2a_ref[...] = aa * d2
  u2b_ref[...] = ab * d2


def _k2(za, zb, degk, deg1, deg2):
  blk = lambda cdim: pl.BlockSpec((BLK, cdim), lambda i: (i, 0))
  return pl.pallas_call(
      _k2_body,
      grid=(GRID,),
      in_specs=[blk(HALF), blk(HALF), blk(16), blk(16), blk(16)],
      out_specs=[blk(HALF)] * 6,
      out_shape=[jax.ShapeDtypeStruct((ROWS, HALF), _f32)] * 6,
  )(za, zb, degk, deg1, deg2)


def _k3_body(x_ref, a1a_ref, a1b_ref, a2a_ref, a2b_ref,
             z11a_ref, z11b_ref, z12a_ref, z12b_ref,
             z21a_ref, z21b_ref, z22a_ref, z22b_ref,
             d1_ref, d2_ref, w2_ref, b2_ref, out_ref):
  d1 = _dinv(d1_ref)
  d2 = _dinv(d2_ref)
  w2 = w2_ref[...]

  def mm(v, c0, w):
    return lax.dot_general(v, w2[:, c0:c0 + w], (((1,), (1,)), ((), ())))

  acc = b2_ref[...] + mm(x_ref[...], 0, NF)
  acc += mm(a1a_ref[...], 128, HALF)
  acc += mm(a1b_ref[...], 192, HALF)
  acc += mm(a2a_ref[...], 256, HALF)
  acc += mm(a2b_ref[...], 320, HALF)
  acc += mm(z11a_ref[...] * d1, 384, HALF)
  acc += mm(z11b_ref[...] * d1, 448, HALF)
  acc += mm(z12a_ref[...] * d1, 512, HALF)
  acc += mm(z12b_ref[...] * d1, 576, HALF)
  acc += mm(z21a_ref[...] * d2, 640, HALF)
  acc += mm(z21b_ref[...] * d2, 704, HALF)
  acc += mm(z22a_ref[...] * d2, 768, HALF)
  acc += mm(z22b_ref[...] * d2, 832, HALF)
  m = jnp.max(acc, axis=1, keepdims=True)
  lse = jnp.log(jnp.sum(jnp.exp(acc - m), axis=1, keepdims=True)) + m
  out_ref[...] = acc - lse


def _k3(x, a1a, a1b, a2a, a2b, zs, deg1, deg2, W2, b2r):
  blk = lambda cdim: pl.BlockSpec((BLK, cdim), lambda i: (i, 0))
  full = lambda shape: pl.BlockSpec(shape, lambda i: (0, 0))
  return pl.pallas_call(
      _k3_body,
      grid=(GRID,),
      in_specs=[blk(NF)] + [blk(HALF)] * 12 + [blk(16), blk(16),
                full((NCLS, 7 * NF)), full((1, NCLS))],
      out_specs=blk(NCLS),
      out_shape=jax.ShapeDtypeStruct((ROWS, NCLS), _f32),
  )(x, a1a, a1b, a2a, a2b, *zs, deg1, deg2, W2, b2r)


def _pad_edges(ei, total):
  pad = total - ei.shape[1]
  fill = jnp.arange(pad, dtype=jnp.int32)
  src = jnp.concatenate([ei[0], fill % N_NODES]).reshape(total // EB, EB)
  dst = jnp.concatenate(
      [ei[1], N_NODES + fill % (ROWS - N_NODES)]).reshape(total // EB, EB)
  return src, dst


def _round_up(e, m):
  return ((e + m - 1) // m) * m


def kernel(h, edge_index, edge_index2, W1, b1, W2, b2):
  chunk = NSUB * EB * SB
  e1p = _round_up(edge_index.shape[1], chunk)
  e2p = _round_up(edge_index2.shape[1], chunk)
  nb1 = e1p // (NSUB * EB)
  nb2 = e2p // (NSUB * EB)
  src1, dst1 = _pad_edges(edge_index, e1p)
  src2, dst2 = _pad_edges(edge_index2, e2p)

  ones_h = jnp.ones((EB, 16), _f32)
  zeros_h = jnp.zeros((ROWS, 16), _f32)
  deg1, deg2 = _deg_kernel(nb1, nb2)(dst1, dst2, ones_h, zeros_h)

  hp = jnp.zeros((ROWS, NF), _f32).at[:N_NODES].set(h)
  b1r = b1.reshape(1, NF)
  b2r = b2.reshape(1, NCLS)

  x, y1a, y1b, y2a, y2b = _k1(hp, W1, b1r, deg1, deg2)

  z1a, z1b, z2a, z2b = _stage1_kernel(nb1, nb2)(
      src1, dst1, src2, dst2, y1a, y1b, y2a, y2b)

  a1a, a1b, u11a, u11b, u21a, u21b = _k2(z1a, z1b, deg1, deg1, deg2)
  a2a, a2b, u12a, u12b, u22a, u22b = _k2(z2a, z2b, deg2, deg1, deg2)

  z11a, z11b, z12a, z12b = _stage2_kernel(nb1)(
      src1, dst1, u11a, u11b, u12a, u12b)
  z21a, z21b, z22a, z22b = _stage2_kernel(nb2)(
      src2, dst2, u21a, u21b, u22a, u22b)
  zs = (z11a, z11b, z12a, z12b, z21a, z21b, z22a, z22b)

  out = _k3(x, a1a, a1b, a2a, a2b, zs, deg1, deg2, W2, b2r)
  return out[:N_NODES]


# speedup vs baseline: 21.0753x; 1.0493x over previous
import functools

import jax
import jax.numpy as jnp
from jax import lax
from jax.experimental import pallas as pl
from jax.experimental.pallas import tpu as pltpu
from jax.experimental.pallas import tpu_sc as plsc

N_NODES = 10000
ROWS = 10240
NF = 128
HALF = 64
NCLS = 40
NCORES = 2
NSUB = 16
EB = 128
SB = 16
NBUF = 4
TILE_ROWS = ROWS // NSUB
BLK = 512
GRID = ROWS // BLK

_f32 = jnp.float32


def _sc_mesh():
  return plsc.VectorSubcoreMesh(
      core_axis_name="c", subcore_axis_name="s",
      num_cores=NCORES, num_subcores=NSUB)


@functools.lru_cache(maxsize=None)
def _deg_kernel(nb1, nb2):
  @functools.partial(
      pl.kernel,
      mesh=_sc_mesh(),
      out_type=[jax.ShapeDtypeStruct((ROWS, 16), _f32),
                jax.ShapeDtypeStruct((ROWS, 16), _f32)],
      scratch_types=[pltpu.VMEM((SB, EB), jnp.int32),
                     pltpu.VMEM((EB, 16), _f32),
                     pltpu.VMEM_SHARED((ROWS, 16), _f32),
                     pltpu.SemaphoreType.DMA],
      compiler_params=pltpu.CompilerParams(use_tc_tiling_on_sc=False),
  )
  def deg_k(dst1, dst2, ones_h, zeros_h, deg1, deg2, didx, ones_v, dacc, sem):
    c = lax.axis_index("c")
    s = lax.axis_index("s")
    sl = pl.ds(s * TILE_ROWS, TILE_ROWS)
    pltpu.sync_copy(ones_h, ones_v)
    pltpu.sync_copy(zeros_h.at[sl], dacc.at[sl])
    plsc.subcore_barrier()

    def run(dst_h, nb):
      def chunk(cc, carry):
        pltpu.sync_copy(dst_h.at[pl.ds(s * nb + cc * SB, SB)], didx)
        descs = [pltpu.async_copy(ones_v, dacc.at[didx.at[b]], sem, add=True)
                 for b in range(SB)]
        for d in descs:
          d.wait()
        return carry
      lax.fori_loop(0, nb // SB, chunk, 0)

    @pl.when(c == 0)
    def _():
      run(dst1, nb1)

    @pl.when(c == 1)
    def _():
      run(dst2, nb2)

    plsc.subcore_barrier()

    @pl.when(c == 0)
    def _():
      pltpu.sync_copy(dacc.at[sl], deg1.at[sl])

    @pl.when(c == 1)
    def _():
      pltpu.sync_copy(dacc.at[sl], deg2.at[sl])

  return deg_k


def _pipe_tables(s, nb, src_h, dst_h, tables, sidx, didx):
  def chunk(cc, carry):
    base = s * nb + cc * SB
    pltpu.sync_copy(src_h.at[pl.ds(base, SB)], sidx)
    pltpu.sync_copy(dst_h.at[pl.ds(base, SB)], didx)
    for (y_h, acc, rows, gsems, ssems) in tables:
      g = {}
      sd = {}
      for j in range(2):
        g[j] = pltpu.async_copy(y_h.at[sidx.at[j]], rows.at[j], gsems[j % 2])
      for b in range(SB):
        if b >= 2 and b + 2 < SB:
          sd[b - 2].wait()
        g[b].wait()
        if b + 2 < SB:
          g[b + 2] = pltpu.async_copy(
              y_h.at[sidx.at[b + 2]], rows.at[(b + 2) % NBUF], gsems[b % 2])
        sd[b] = pltpu.async_copy(rows.at[b % NBUF], acc.at[didx.at[b]],
                                 ssems[b % 2], add=True)
      for j in range(max(0, SB - 4), SB):
        sd[j].wait()
    return carry
  lax.fori_loop(0, nb // SB, chunk, 0)


def _init_acc(c, sl, ya_h, yb_h, acc):
  @pl.when(c == 0)
  def _():
    pltpu.sync_copy(ya_h.at[sl], acc.at[sl])

  @pl.when(c == 1)
  def _():
    pltpu.sync_copy(yb_h.at[sl], acc.at[sl])


def _copy_out(c, sl, acc, za_h, zb_h):
  @pl.when(c == 0)
  def _():
    pltpu.sync_copy(acc.at[sl], za_h.at[sl])

  @pl.when(c == 1)
  def _():
    pltpu.sync_copy(acc.at[sl], zb_h.at[sl])


@functools.lru_cache(maxsize=None)
def _stage1_kernel(nb1, nb2):
  @functools.partial(
      pl.kernel,
      mesh=_sc_mesh(),
      out_type=[jax.ShapeDtypeStruct((ROWS, HALF), _f32)] * 4,
      scratch_types=[pltpu.VMEM((SB, EB), jnp.int32),
                     pltpu.VMEM((SB, EB), jnp.int32),
                     pltpu.VMEM((NBUF, EB, HALF), _f32),
                     pltpu.VMEM_SHARED((ROWS, HALF), _f32),
                     pltpu.VMEM_SHARED((ROWS, HALF), _f32)] +
                    [pltpu.SemaphoreType.DMA] * 4,
      compiler_params=pltpu.CompilerParams(use_tc_tiling_on_sc=False),
  )
  def k(src1, dst1, src2, dst2, y1a, y1b, y2a, y2b,
        z1a, z1b, z2a, z2b, sidx, didx, rows, acc1, acc2, g0, g1, s0, s1):
    c = lax.axis_index("c")
    s = lax.axis_index("s")
    sl = pl.ds(s * TILE_ROWS, TILE_ROWS)
    _init_acc(c, sl, y1a, y1b, acc1)
    _init_acc(c, sl, y2a, y2b, acc2)
    plsc.subcore_barrier()

    @pl.when(c == 0)
    def _():
      _pipe_tables(s, nb1, src1, dst1,
                   [(y1a, acc1, rows, (g0, g1), (s0, s1))], sidx, didx)
      _pipe_tables(s, nb2, src2, dst2,
                   [(y2a, acc2, rows, (g0, g1), (s0, s1))], sidx, didx)

    @pl.when(c == 1)
    def _():
      _pipe_tables(s, nb1, src1, dst1,
                   [(y1b, acc1, rows, (g0, g1), (s0, s1))], sidx, didx)
      _pipe_tables(s, nb2, src2, dst2,
                   [(y2b, acc2, rows, (g0, g1), (s0, s1))], sidx, didx)

    plsc.subcore_barrier()
    _copy_out(c, sl, acc1, z1a, z1b)
    _copy_out(c, sl, acc2, z2a, z2b)

  return k


@functools.lru_cache(maxsize=None)
def _stage2_kernel(nb):
  @functools.partial(
      pl.kernel,
      mesh=_sc_mesh(),
      out_type=[jax.ShapeDtypeStruct((ROWS, NF), _f32)] * 2,
      scratch_types=[pltpu.VMEM((SB, EB), jnp.int32),
                     pltpu.VMEM((SB, EB), jnp.int32),
                     pltpu.VMEM((2, EB, NF), _f32),
                     pltpu.VMEM_SHARED((ROWS, NF), _f32)] +
                    [pltpu.SemaphoreType.DMA] * 4,
      compiler_params=pltpu.CompilerParams(use_tc_tiling_on_sc=False),
  )
  def k(src_h, dst_h, t0, t1, zc0, zc1, sidx, didx, rows, acc, g0, g1, s0, s1):
    c = lax.axis_index("c")
    s = lax.axis_index("s")
    sl = pl.ds(s * TILE_ROWS, TILE_ROWS)
    _init_acc(c, sl, t0, t1, acc)
    plsc.subcore_barrier()

    def run(t_h):
      def chunk(cc, carry):
        base = s * nb + cc * SB
        pltpu.sync_copy(src_h.at[pl.ds(base, SB)], sidx)
        pltpu.sync_copy(dst_h.at[pl.ds(base, SB)], didx)
        g = {0: pltpu.async_copy(t_h.at[sidx.at[0]], rows.at[0], g0)}
        sd = {}
        for b in range(SB):
          if b >= 1:
            sd[b - 1].wait()
          g[b].wait()
          if b + 1 < SB:
            g[b + 1] = pltpu.async_copy(
                t_h.at[sidx.at[b + 1]], rows.at[(b + 1) % 2],
                g0 if (b + 1) % 2 == 0 else g1)
          sd[b] = pltpu.async_copy(rows.at[b % 2], acc.at[didx.at[b]],
                                   s0 if b % 2 == 0 else s1, add=True)
        sd[SB - 1].wait()
        return carry
      lax.fori_loop(0, nb // SB, chunk, 0)

    @pl.when(c == 0)
    def _():
      run(t0)

    @pl.when(c == 1)
    def _():
      run(t1)

    plsc.subcore_barrier()
    _copy_out(c, sl, acc, zc0, zc1)

  return k


def _dinv(deg_blk):
  return lax.rsqrt(deg_blk[:, 0:1] + 1.0)


def _k1_body(h_ref, w_ref, b_ref, d1_ref, d2_ref,
             x_ref, y1a_ref, y1b_ref, y2a_ref, y2b_ref):
  xb = lax.dot_general(h_ref[...], w_ref[...], (((1,), (1,)), ((), ())))
  xb = jnp.maximum(xb + b_ref[...], 0.0)
  x_ref[...] = xb
  y1 = xb * _dinv(d1_ref)
  y2 = xb * _dinv(d2_ref)
  y1a_ref[...] = y1[:, :HALF]
  y1b_ref[...] = y1[:, HALF:]
  y2a_ref[...] = y2[:, :HALF]
  y2b_ref[...] = y2[:, HALF:]


def _k1(hp, W1, b1r, deg1, deg2):
  blk = lambda r, cdim: pl.BlockSpec((r, cdim), lambda i: (i, 0))
  full = lambda shape: pl.BlockSpec(shape, lambda i: (0, 0))
  return pl.pallas_call(
      _k1_body,
      grid=(GRID,),
      in_specs=[blk(BLK, NF), full((NF, NF)), full((1, NF)),
                blk(BLK, 16), blk(BLK, 16)],
      out_specs=[blk(BLK, NF)] + [blk(BLK, HALF)] * 4,
      out_shape=[jax.ShapeDtypeStruct((ROWS, NF), _f32)] +
                [jax.ShapeDtypeStruct((ROWS, HALF), _f32)] * 4,
  )(hp, W1, b1r, deg1, deg2)


def _k2_body(z1a_ref, z1b_ref, z2a_ref, z2b_ref, d1_ref, d2_ref,
             a1a_ref, a1b_ref, a2a_ref, a2b_ref,
             t10_ref, t11_ref, t20_ref, t21_ref):
  d1 = _dinv(d1_ref)
  d2 = _dinv(d2_ref)
  a1a = z1a_ref[...] * d1
  a1b = z1b_ref[...] * d1
  a2a = z2a_ref[...] * d2
  a2b = z2b_ref[...] * d2
  a1a_ref[...] = a1a
  a1b_ref[...] = a1b
  a2a_ref[...] = a2a
  a2b_ref[...] = a2b
  t10_ref[...] = jnp.concatenate([a1a * d1, a2a * d1], axis=1)
  t11_ref[...] = jnp.concatenate([a1b * d1, a2b * d1], axis=1)
  t20_ref[...] = jnp.concatenate([a1a * d2, a2a * d2], axis=1)
  t21_ref[...] = jnp.concatenate([a1b * d2, a2b * d2], axis=1)


def _k2(z1a, z1b, z2a, z2b, deg1, deg2):
  blk = lambda cdim: pl.BlockSpec((BLK, cdim), lambda i: (i, 0))
  return pl.pallas_call(
      _k2_body,
      grid=(GRID,),
      in_specs=[blk(HALF)] * 4 + [blk(16), blk(16)],
      out_specs=[blk(HALF)] * 4 + [blk(NF)] * 4,
      out_shape=[jax.ShapeDtypeStruct((ROWS, HALF), _f32)] * 4 +
                [jax.ShapeDtypeStruct((ROWS, NF), _f32)] * 4,
  )(z1a, z1b, z2a, z2b, deg1, deg2)


def _k3_body(x_ref, a1a_ref, a1b_ref, a2a_ref, a2b_ref,
             zc10_ref, zc11_ref, zc20_ref, zc21_ref,
             d1_ref, d2_ref, w2_ref, b2_ref, out_ref):
  d1 = _dinv(d1_ref)
  d2 = _dinv(d2_ref)
  w2 = w2_ref[...]

  def mm(v, c0, w):
    return lax.dot_general(v, w2[:, c0:c0 + w], (((1,), (1,)), ((), ())))

  zc10 = zc10_ref[...] * d1
  zc11 = zc11_ref[...] * d1
  zc20 = zc20_ref[...] * d2
  zc21 = zc21_ref[...] * d2
  acc = b2_ref[...] + mm(x_ref[...], 0, NF)
  acc += mm(a1a_ref[...], 128, HALF)
  acc += mm(a1b_ref[...], 192, HALF)
  acc += mm(a2a_ref[...], 256, HALF)
  acc += mm(a2b_ref[...], 320, HALF)
  acc += mm(zc10[:, :HALF], 384, HALF)
  acc += mm(zc11[:, :HALF], 448, HALF)
  acc += mm(zc10[:, HALF:], 512, HALF)
  acc += mm(zc11[:, HALF:], 576, HALF)
  acc += mm(zc20[:, :HALF], 640, HALF)
  acc += mm(zc21[:, :HALF], 704, HALF)
  acc += mm(zc20[:, HALF:], 768, HALF)
  acc += mm(zc21[:, HALF:], 832, HALF)
  m = jnp.max(acc, axis=1, keepdims=True)
  lse = jnp.log(jnp.sum(jnp.exp(acc - m), axis=1, keepdims=True)) + m
  out_ref[...] = acc - lse


def _k3(x, a1a, a1b, a2a, a2b, zcs, deg1, deg2, W2, b2r):
  blk = lambda cdim: pl.BlockSpec((BLK, cdim), lambda i: (i, 0))
  full = lambda shape: pl.BlockSpec(shape, lambda i: (0, 0))
  return pl.pallas_call(
      _k3_body,
      grid=(GRID,),
      in_specs=[blk(NF)] + [blk(HALF)] * 4 + [blk(NF)] * 4 +
               [blk(16), blk(16), full((NCLS, 7 * NF)), full((1, NCLS))],
      out_specs=blk(NCLS),
      out_shape=jax.ShapeDtypeStruct((ROWS, NCLS), _f32),
  )(x, a1a, a1b, a2a, a2b, *zcs, deg1, deg2, W2, b2r)


def _pad_edges(ei, total):
  pad = total - ei.shape[1]
  fill = jnp.arange(pad, dtype=jnp.int32)
  src = jnp.concatenate([ei[0], fill % N_NODES]).reshape(total // EB, EB)
  dst = jnp.concatenate(
      [ei[1], N_NODES + fill % (ROWS - N_NODES)]).reshape(total // EB, EB)
  return src, dst


def _round_up(e, m):
  return ((e + m - 1) // m) * m


def kernel(h, edge_index, edge_index2, W1, b1, W2, b2):
  chunk = NSUB * EB * SB
  e1p = _round_up(edge_index.shape[1], chunk)
  e2p = _round_up(edge_index2.shape[1], chunk)
  nb1 = e1p // (NSUB * EB)
  nb2 = e2p // (NSUB * EB)
  src1, dst1 = _pad_edges(edge_index, e1p)
  src2, dst2 = _pad_edges(edge_index2, e2p)

  ones_h = jnp.ones((EB, 16), _f32)
  zeros_h = jnp.zeros((ROWS, 16), _f32)
  deg1, deg2 = _deg_kernel(nb1, nb2)(dst1, dst2, ones_h, zeros_h)

  hp = jnp.zeros((ROWS, NF), _f32).at[:N_NODES].set(h)
  b1r = b1.reshape(1, NF)
  b2r = b2.reshape(1, NCLS)

  x, y1a, y1b, y2a, y2b = _k1(hp, W1, b1r, deg1, deg2)

  z1a, z1b, z2a, z2b = _stage1_kernel(nb1, nb2)(
      src1, dst1, src2, dst2, y1a, y1b, y2a, y2b)

  a1a, a1b, a2a, a2b, t10, t11, t20, t21 = _k2(z1a, z1b, z2a, z2b, deg1, deg2)

  zc10, zc11 = _stage2_kernel(nb1)(src1, dst1, t10, t11)
  zc20, zc21 = _stage2_kernel(nb2)(src2, dst2, t20, t21)

  out = _k3(x, a1a, a1b, a2a, a2b, (zc10, zc11, zc20, zc21),
            deg1, deg2, W2, b2r)
  return out[:N_NODES]


# speedup vs baseline: 22.2640x; 1.0564x over previous
import functools

import jax
import jax.numpy as jnp
from jax import lax
from jax.experimental import pallas as pl
from jax.experimental.pallas import tpu as pltpu
from jax.experimental.pallas import tpu_sc as plsc

N_NODES = 10000
ROWS = 10240
NF = 128
HALF = 64
NCLS = 40
NCORES = 2
NSUB = 16
EB = 128
SB = 20
NBUF = 4
TILE_ROWS = ROWS // NSUB
BLK = 512
GRID = ROWS // BLK

_f32 = jnp.float32


def _sc_mesh():
  return plsc.VectorSubcoreMesh(
      core_axis_name="c", subcore_axis_name="s",
      num_cores=NCORES, num_subcores=NSUB)


@functools.lru_cache(maxsize=None)
def _deg_kernel(nb1, nb2):
  @functools.partial(
      pl.kernel,
      mesh=_sc_mesh(),
      out_type=[jax.ShapeDtypeStruct((ROWS, 16), _f32),
                jax.ShapeDtypeStruct((ROWS, 16), _f32)],
      scratch_types=[pltpu.VMEM((SB, EB), jnp.int32),
                     pltpu.VMEM((EB, 16), _f32),
                     pltpu.VMEM_SHARED((ROWS, 16), _f32),
                     pltpu.SemaphoreType.DMA],
      compiler_params=pltpu.CompilerParams(use_tc_tiling_on_sc=False),
  )
  def deg_k(dst1, dst2, ones_h, zeros_h, deg1, deg2, didx, ones_v, dacc, sem):
    c = lax.axis_index("c")
    s = lax.axis_index("s")
    sl = pl.ds(s * TILE_ROWS, TILE_ROWS)
    pltpu.sync_copy(ones_h, ones_v)
    pltpu.sync_copy(zeros_h.at[sl], dacc.at[sl])
    plsc.subcore_barrier()

    def run(dst_h, nb):
      def chunk(cc, carry):
        pltpu.sync_copy(dst_h.at[pl.ds(s * nb + cc * SB, SB)], didx)
        descs = [pltpu.async_copy(ones_v, dacc.at[didx.at[b]], sem, add=True)
                 for b in range(SB)]
        for d in descs:
          d.wait()
        return carry
      lax.fori_loop(0, nb // SB, chunk, 0)

    @pl.when(c == 0)
    def _():
      run(dst1, nb1)

    @pl.when(c == 1)
    def _():
      run(dst2, nb2)

    plsc.subcore_barrier()

    @pl.when(c == 0)
    def _():
      pltpu.sync_copy(dacc.at[sl], deg1.at[sl])

    @pl.when(c == 1)
    def _():
      pltpu.sync_copy(dacc.at[sl], deg2.at[sl])

  return deg_k


def _pipe_edges(s, nb, src_h, dst_h, y_h, acc, sidx, didx, rows,
                gsems, ssems, isems, nbuf):
  nch = nb // SB
  assert nch % 2 == 0
  dep = nbuf // 2

  def load_idx(cc, p):
    base = s * nb + cc * SB
    pltpu.async_copy(src_h.at[pl.ds(base, SB)], sidx.at[p], isems[p])
    pltpu.async_copy(dst_h.at[pl.ds(base, SB)], didx.at[p], isems[p])

  def drain_idx(p):
    pltpu.make_async_copy(src_h.at[pl.ds(0, SB)], sidx.at[p], isems[p]).wait()
    pltpu.make_async_copy(dst_h.at[pl.ds(0, SB)], didx.at[p], isems[p]).wait()

  def do_chunk(p):
    g = {}
    sd = {}
    for j in range(dep):
      g[j] = pltpu.async_copy(y_h.at[sidx.at[p, j]], rows.at[j % nbuf],
                              gsems[j % 2])
    for b in range(SB):
      if b >= dep:
        sd[b - dep].wait()
      g[b].wait()
      if b + dep < SB:
        g[b + dep] = pltpu.async_copy(
            y_h.at[sidx.at[p, b + dep]], rows.at[(b + dep) % nbuf],
            gsems[(b + dep) % 2])
      sd[b] = pltpu.async_copy(rows.at[b % nbuf], acc.at[didx.at[p, b]],
                               ssems[b % 2], add=True)
    for j in range(SB - dep, SB):
      sd[j].wait()

  load_idx(0, 0)

  def body(i, carry):
    cc = 2 * i
    drain_idx(0)
    load_idx(jnp.minimum(cc + 1, nch - 1), 1)
    do_chunk(0)
    drain_idx(1)
    load_idx(jnp.minimum(cc + 2, nch - 1), 0)
    do_chunk(1)
    return carry

  lax.fori_loop(0, nch // 2, body, 0)
  drain_idx(0)


def _init_acc(c, sl, ya_h, yb_h, acc):
  @pl.when(c == 0)
  def _():
    pltpu.sync_copy(ya_h.at[sl], acc.at[sl])

  @pl.when(c == 1)
  def _():
    pltpu.sync_copy(yb_h.at[sl], acc.at[sl])


def _copy_out(c, sl, acc, za_h, zb_h):
  @pl.when(c == 0)
  def _():
    pltpu.sync_copy(acc.at[sl], za_h.at[sl])

  @pl.when(c == 1)
  def _():
    pltpu.sync_copy(acc.at[sl], zb_h.at[sl])


@functools.lru_cache(maxsize=None)
def _stage1_kernel(nb1, nb2):
  @functools.partial(
      pl.kernel,
      mesh=_sc_mesh(),
      out_type=[jax.ShapeDtypeStruct((ROWS, HALF), _f32)] * 4,
      scratch_types=[pltpu.VMEM((2, SB, EB), jnp.int32),
                     pltpu.VMEM((2, SB, EB), jnp.int32),
                     pltpu.VMEM((NBUF, EB, HALF), _f32),
                     pltpu.VMEM_SHARED((ROWS, HALF), _f32),
                     pltpu.VMEM_SHARED((ROWS, HALF), _f32)] +
                    [pltpu.SemaphoreType.DMA] * 6,
      compiler_params=pltpu.CompilerParams(use_tc_tiling_on_sc=False),
  )
  def k(src1, dst1, src2, dst2, y1a, y1b, y2a, y2b,
        z1a, z1b, z2a, z2b, sidx, didx, rows, acc1, acc2,
        g0, g1, s0, s1, i0, i1):
    c = lax.axis_index("c")
    s = lax.axis_index("s")
    sl = pl.ds(s * TILE_ROWS, TILE_ROWS)
    _init_acc(c, sl, y1a, y1b, acc1)
    _init_acc(c, sl, y2a, y2b, acc2)
    plsc.subcore_barrier()

    def run(y1_h, y2_h):
      _pipe_edges(s, nb1, src1, dst1, y1_h, acc1, sidx, didx, rows,
                  (g0, g1), (s0, s1), (i0, i1), NBUF)
      _pipe_edges(s, nb2, src2, dst2, y2_h, acc2, sidx, didx, rows,
                  (g0, g1), (s0, s1), (i0, i1), NBUF)

    @pl.when(c == 0)
    def _():
      run(y1a, y2a)

    @pl.when(c == 1)
    def _():
      run(y1b, y2b)

    plsc.subcore_barrier()
    _copy_out(c, sl, acc1, z1a, z1b)
    _copy_out(c, sl, acc2, z2a, z2b)

  return k


@functools.lru_cache(maxsize=None)
def _stage2_kernel(nb):
  @functools.partial(
      pl.kernel,
      mesh=_sc_mesh(),
      out_type=[jax.ShapeDtypeStruct((ROWS, NF), _f32)] * 2,
      scratch_types=[pltpu.VMEM((2, SB, EB), jnp.int32),
                     pltpu.VMEM((2, SB, EB), jnp.int32),
                     pltpu.VMEM((2, EB, NF), _f32),
                     pltpu.VMEM_SHARED((ROWS, NF), _f32)] +
                    [pltpu.SemaphoreType.DMA] * 6,
      compiler_params=pltpu.CompilerParams(use_tc_tiling_on_sc=False),
  )
  def k(src_h, dst_h, t0, t1, zc0, zc1, sidx, didx, rows, acc,
        g0, g1, s0, s1, i0, i1):
    c = lax.axis_index("c")
    s = lax.axis_index("s")
    sl = pl.ds(s * TILE_ROWS, TILE_ROWS)
    _init_acc(c, sl, t0, t1, acc)
    plsc.subcore_barrier()

    def run(t_h):
      _pipe_edges(s, nb, src_h, dst_h, t_h, acc, sidx, didx, rows,
                  (g0, g1), (s0, s1), (i0, i1), 2)

    @pl.when(c == 0)
    def _():
      run(t0)

    @pl.when(c == 1)
    def _():
      run(t1)

    plsc.subcore_barrier()
    _copy_out(c, sl, acc, zc0, zc1)

  return k


def _dinv(deg_blk):
  return lax.rsqrt(deg_blk[:, 0:1] + 1.0)


def _k1_body(h_ref, w_ref, b_ref, d1_ref, d2_ref,
             x_ref, y1a_ref, y1b_ref, y2a_ref, y2b_ref):
  xb = lax.dot_general(h_ref[...], w_ref[...], (((1,), (1,)), ((), ())))
  xb = jnp.maximum(xb + b_ref[...], 0.0)
  x_ref[...] = xb
  y1 = xb * _dinv(d1_ref)
  y2 = xb * _dinv(d2_ref)
  y1a_ref[...] = y1[:, :HALF]
  y1b_ref[...] = y1[:, HALF:]
  y2a_ref[...] = y2[:, :HALF]
  y2b_ref[...] = y2[:, HALF:]


def _k1(hp, W1, b1r, deg1, deg2):
  blk = lambda r, cdim: pl.BlockSpec((r, cdim), lambda i: (i, 0))
  full = lambda shape: pl.BlockSpec(shape, lambda i: (0, 0))
  return pl.pallas_call(
      _k1_body,
      grid=(GRID,),
      in_specs=[blk(BLK, NF), full((NF, NF)), full((1, NF)),
                blk(BLK, 16), blk(BLK, 16)],
      out_specs=[blk(BLK, NF)] + [blk(BLK, HALF)] * 4,
      out_shape=[jax.ShapeDtypeStruct((ROWS, NF), _f32)] +
                [jax.ShapeDtypeStruct((ROWS, HALF), _f32)] * 4,
  )(hp, W1, b1r, deg1, deg2)


def _k2_body(z1a_ref, z1b_ref, z2a_ref, z2b_ref, d1_ref, d2_ref,
             t10_ref, t11_ref, t20_ref, t21_ref):
  d1 = _dinv(d1_ref)
  d2 = _dinv(d2_ref)
  a1a = z1a_ref[...] * d1
  a1b = z1b_ref[...] * d1
  a2a = z2a_ref[...] * d2
  a2b = z2b_ref[...] * d2
  t10_ref[...] = jnp.concatenate([a1a * d1, a2a * d1], axis=1)
  t11_ref[...] = jnp.concatenate([a1b * d1, a2b * d1], axis=1)
  t20_ref[...] = jnp.concatenate([a1a * d2, a2a * d2], axis=1)
  t21_ref[...] = jnp.concatenate([a1b * d2, a2b * d2], axis=1)


def _k2(z1a, z1b, z2a, z2b, deg1, deg2):
  blk = lambda cdim: pl.BlockSpec((BLK, cdim), lambda i: (i, 0))
  return pl.pallas_call(
      _k2_body,
      grid=(GRID,),
      in_specs=[blk(HALF)] * 4 + [blk(16), blk(16)],
      out_specs=[blk(NF)] * 4,
      out_shape=[jax.ShapeDtypeStruct((ROWS, NF), _f32)] * 4,
  )(z1a, z1b, z2a, z2b, deg1, deg2)


def _k3_body(x_ref, z1a_ref, z1b_ref, z2a_ref, z2b_ref,
             zc10_ref, zc11_ref, zc20_ref, zc21_ref,
             d1_ref, d2_ref, w2_ref, b2_ref, out_ref):
  d1 = _dinv(d1_ref)
  d2 = _dinv(d2_ref)
  w2 = w2_ref[...]

  def mm(v, c0, w):
    return lax.dot_general(v, w2[:, c0:c0 + w], (((1,), (1,)), ((), ())))

  zc10 = zc10_ref[...] * d1
  zc11 = zc11_ref[...] * d1
  zc20 = zc20_ref[...] * d2
  zc21 = zc21_ref[...] * d2
  acc = b2_ref[...] + mm(x_ref[...], 0, NF)
  acc += mm(z1a_ref[...] * d1, 128, HALF)
  acc += mm(z1b_ref[...] * d1, 192, HALF)
  acc += mm(z2a_ref[...] * d2, 256, HALF)
  acc += mm(z2b_ref[...] * d2, 320, HALF)
  acc += mm(zc10[:, :HALF], 384, HALF)
  acc += mm(zc11[:, :HALF], 448, HALF)
  acc += mm(zc10[:, HALF:], 512, HALF)
  acc += mm(zc11[:, HALF:], 576, HALF)
  acc += mm(zc20[:, :HALF], 640, HALF)
  acc += mm(zc21[:, :HALF], 704, HALF)
  acc += mm(zc20[:, HALF:], 768, HALF)
  acc += mm(zc21[:, HALF:], 832, HALF)
  m = jnp.max(acc, axis=1, keepdims=True)
  lse = jnp.log(jnp.sum(jnp.exp(acc - m), axis=1, keepdims=True)) + m
  out_ref[...] = acc - lse


def _k3(x, z1a, z1b, z2a, z2b, zcs, deg1, deg2, W2, b2r):
  blk = lambda cdim: pl.BlockSpec((BLK, cdim), lambda i: (i, 0))
  full = lambda shape: pl.BlockSpec(shape, lambda i: (0, 0))
  return pl.pallas_call(
      _k3_body,
      grid=(GRID,),
      in_specs=[blk(NF)] + [blk(HALF)] * 4 + [blk(NF)] * 4 +
               [blk(16), blk(16), full((NCLS, 7 * NF)), full((1, NCLS))],
      out_specs=blk(NCLS),
      out_shape=jax.ShapeDtypeStruct((ROWS, NCLS), _f32),
  )(x, z1a, z1b, z2a, z2b, *zcs, deg1, deg2, W2, b2r)


def _pad_edges(ei, total):
  pad = total - ei.shape[1]
  fill = jnp.arange(pad, dtype=jnp.int32)
  src = jnp.concatenate([ei[0], fill % N_NODES]).reshape(total // EB, EB)
  dst = jnp.concatenate(
      [ei[1], N_NODES + fill % (ROWS - N_NODES)]).reshape(total // EB, EB)
  return src, dst


def _round_up(e, m):
  return ((e + m - 1) // m) * m


def kernel(h, edge_index, edge_index2, W1, b1, W2, b2):
  chunk = NSUB * EB * SB * 2
  e1p = _round_up(edge_index.shape[1], chunk)
  e2p = _round_up(edge_index2.shape[1], chunk)
  nb1 = e1p // (NSUB * EB)
  nb2 = e2p // (NSUB * EB)
  src1, dst1 = _pad_edges(edge_index, e1p)
  src2, dst2 = _pad_edges(edge_index2, e2p)

  ones_h = jnp.ones((EB, 16), _f32)
  zeros_h = jnp.zeros((ROWS, 16), _f32)
  deg1, deg2 = _deg_kernel(nb1, nb2)(dst1, dst2, ones_h, zeros_h)

  hp = jnp.zeros((ROWS, NF), _f32).at[:N_NODES].set(h)
  b1r = b1.reshape(1, NF)
  b2r = b2.reshape(1, NCLS)

  x, y1a, y1b, y2a, y2b = _k1(hp, W1, b1r, deg1, deg2)

  z1a, z1b, z2a, z2b = _stage1_kernel(nb1, nb2)(
      src1, dst1, src2, dst2, y1a, y1b, y2a, y2b)

  t10, t11, t20, t21 = _k2(z1a, z1b, z2a, z2b, deg1, deg2)

  zc10, zc11 = _stage2_kernel(nb1)(src1, dst1, t10, t11)
  zc20, zc21 = _stage2_kernel(nb2)(src2, dst2, t20, t21)

  out = _k3(x, z1a, z1b, z2a, z2b, (zc10, zc11, zc20, zc21),
            deg1, deg2, W2, b2r)
  return out[:N_NODES]
